# baseline (speedup 1.0000x reference)
import dataclasses

import jax
import jax.numpy as jnp
from jax import lax
from jax.experimental import pallas as pl
from jax.experimental.pallas import tpu as pltpu
from jax.experimental.pallas import tpu_sc as plsc

NC = 2
NS = 16
LANES = 16
CHUNK = 400
SB = 200
NSTREAM = CHUNK // SB
NCPS = 250
RING = 6
NFULL = (NCPS // RING) * RING
ZROWS = 400
NPAD = 102400
NR128 = NPAD // 8
NCH = 1600000 // CHUNK


def _sc_compiler_params():
    cp = pltpu.CompilerParams()
    fields = pltpu.CompilerParams.__dataclass_fields__
    if "needs_layout_passes" in fields:
        cp = dataclasses.replace(cp, needs_layout_passes=False)
    if "use_tc_tiling_on_sc" in fields:
        cp = dataclasses.replace(cp, use_tc_tiling_on_sc=False)
    return cp


def _gcn_body(dst_hbm, src_hbm, w_hbm, tab_hbm, out_hbm,
              dst_v, src_v, w_v, rows, acc,
              si0, si1, si2, sg0, sg1, ss0, ss1):
    c = lax.axis_index("c")
    s = lax.axis_index("s")
    sems_i = (si0, si1, si2)
    sems_g = (sg0, sg1)
    sems_s = (ss0, ss1)
    rows_per_sub = NPAD // NS

    def zero_acc_stripe():
        @plsc.parallel_loop(0, ZROWS, unroll=4)
        def _zero_buf(i):
            rows[0, i, :] = jnp.zeros((LANES,), jnp.float32)

        @pl.loop(0, rows_per_sub // ZROWS)
        def _zero_acc(k):
            pltpu.sync_copy(rows.at[0].at[pl.ds(0, ZROWS)],
                            acc.at[pl.ds(s * rows_per_sub + k * ZROWS, ZROWS)])

    def chunk_id(ci):
        return ci * NS + s

    def fire_idx(ci, t):
        ch = chunk_id(ci)
        pltpu.async_copy(dst_hbm.at[ch], dst_v.at[t], sems_i[t])
        pltpu.async_copy(src_hbm.at[ch], src_v.at[t], sems_i[t])
        pltpu.async_copy(w_hbm.at[ch], w_v.at[t], sems_i[t])

    def wait_idx(ci, t):
        ch = chunk_id(ci)
        pltpu.make_async_copy(dst_hbm.at[ch], dst_v.at[t], sems_i[t]).wait()
        pltpu.make_async_copy(src_hbm.at[ch], src_v.at[t], sems_i[t]).wait()
        pltpu.make_async_copy(w_hbm.at[ch], w_v.at[t], sems_i[t]).wait()

    def fire_scatter(t, r):
        for j in range(NSTREAM):
            pltpu.async_copy(rows.at[r].at[pl.ds(j * SB, SB)],
                             acc.at[dst_v.at[t].at[j]], sems_s[r], add=True)

    def wait_scatter(t, r):
        for j in range(NSTREAM):
            pltpu.make_async_copy(rows.at[r].at[pl.ds(j * SB, SB)],
                                  acc.at[dst_v.at[t].at[j]],
                                  sems_s[r]).wait()

    def run_pass(tab, do_relu, out_slot):
        def fire_gather(t, r):
            for j in range(NSTREAM):
                pltpu.async_copy(tab.at[src_v.at[t].at[j]],
                                 rows.at[r].at[pl.ds(j * SB, SB)], sems_g[r])

        def wait_gather(t, r):
            for j in range(NSTREAM):
                pltpu.make_async_copy(tab.at[src_v.at[t].at[j]],
                                      rows.at[r].at[pl.ds(j * SB, SB)],
                                      sems_g[r]).wait()

        def multiply(t, r):
            rv = rows.at[r]
            wv = w_v.at[t]

            @plsc.parallel_loop(0, CHUNK // LANES)
            def _grp(g):
                base = g * LANES
                wgrp = wv[pl.ds(base, LANES)]
                for i in range(LANES):
                    wb = lax.gather(
                        wgrp, jnp.full((LANES, 1), i, jnp.int32),
                        dimension_numbers=lax.GatherDimensionNumbers(
                            offset_dims=(), collapsed_slice_dims=(0,),
                            start_index_map=(0,)),
                        slice_sizes=(1,),
                        mode=lax.GatherScatterMode.PROMISE_IN_BOUNDS)
                    if do_relu:
                        rows[r, base + i, :] = (
                            jnp.maximum(rv[base + i, :], 0.0) * wb)
                    else:
                        rows[r, base + i, :] = rv[base + i, :] * wb

        def ring_step(ci, k):
            t, tp1, tp2 = k % 3, (k + 1) % 3, (k + 2) % 3
            r, rp1 = k % 2, (k + 1) % 2

            @pl.when(ci > 0)
            def _():
                wait_scatter(tp2, rp1)

            @pl.when(ci + 1 < NCPS)
            def _():
                wait_idx(ci + 1, tp1)
                fire_gather(tp1, rp1)

            @pl.when(ci + 2 < NCPS)
            def _():
                fire_idx(ci + 2, tp2)

            wait_gather(t, r)
            multiply(t, r)
            fire_scatter(t, r)

        fire_idx(0, 0)
        fire_idx(1, 1)
        wait_idx(0, 0)
        fire_gather(0, 0)

        @pl.loop(0, NFULL // RING)
        def _ring(rb):
            base = rb * RING
            for k in range(RING):
                ring_step(base + k, k)

        for ci in range(NFULL, NCPS):
            ring_step(ci, ci % RING)

        wait_scatter((NCPS - 1) % 3, (NCPS - 1) % 2)

        plsc.subcore_barrier()
        pltpu.sync_copy(
            acc.at[pl.ds(s * rows_per_sub, rows_per_sub)],
            out_hbm.at[out_slot].at[c].at[pl.ds(s * rows_per_sub,
                                                rows_per_sub)])

    zero_acc_stripe()
    plsc.subcore_barrier()
    run_pass(tab_hbm.at[c], False, 0)
    zero_acc_stripe()
    plsc.subcore_barrier()
    run_pass(out_hbm.at[0].at[c], True, 1)


def _gcn_sc(dst_r, src_r, w_r, tab):
    mesh = plsc.VectorSubcoreMesh(core_axis_name="c", subcore_axis_name="s")
    k = pl.kernel(
        _gcn_body,
        out_type=jax.ShapeDtypeStruct((2, NC, NPAD, LANES), jnp.float32),
        mesh=mesh,
        scratch_types=[
            pltpu.VMEM((3, NSTREAM, SB), jnp.int32),
            pltpu.VMEM((3, NSTREAM, SB), jnp.int32),
            pltpu.VMEM((3, CHUNK), jnp.float32),
            pltpu.VMEM((2, CHUNK, LANES), jnp.float32),
            pltpu.VMEM_SHARED((NPAD, LANES), jnp.float32),
            pltpu.SemaphoreType.DMA,
            pltpu.SemaphoreType.DMA,
            pltpu.SemaphoreType.DMA,
            pltpu.SemaphoreType.DMA,
            pltpu.SemaphoreType.DMA,
            pltpu.SemaphoreType.DMA,
            pltpu.SemaphoreType.DMA,
        ],
        compiler_params=_sc_compiler_params(),
    )
    return k(dst_r, src_r, w_r, tab)


def _finish_body(x_ref, m_ref, o_ref):
    hp = lax.Precision.HIGHEST
    o_ref[...] = (jnp.dot(x_ref[0], m_ref[0], precision=hp)
                  + jnp.dot(x_ref[1], m_ref[1], precision=hp))


def _finish(v128, w2):
    ncls = w2.shape[1]
    wpad = jnp.pad(w2, ((0, 0), (0, 2 * LANES - ncls)))
    eye8 = jnp.eye(8, dtype=jnp.float32)
    mats = jnp.stack([jnp.kron(eye8, wpad[:LANES, :]),
                      jnp.kron(eye8, wpad[LANES:, :])])
    bn = 3200
    grid = NR128 // bn
    return pl.pallas_call(
        _finish_body,
        grid=(grid,),
        in_specs=[
            pl.BlockSpec((NC, bn, 128), lambda i: (0, i, 0)),
            pl.BlockSpec((2, 128, 256), lambda i: (0, 0, 0)),
        ],
        out_specs=pl.BlockSpec((bn, 256), lambda i: (i, 0)),
        out_shape=jax.ShapeDtypeStruct((NR128, 256), jnp.float32),
    )(v128, mats)


def kernel(edge_index, edge_weight, W1, W2):
    n = W1.shape[0]
    dst_r = edge_index[0].astype(jnp.int32).reshape(NCH, NSTREAM, SB)
    src_r = edge_index[1].astype(jnp.int32).reshape(NCH, NSTREAM, SB)
    w_r = edge_weight.reshape(NCH, CHUNK)
    tab1 = W1.reshape(n, NC, LANES).transpose(1, 0, 2)
    hv = _gcn_sc(dst_r, src_r, w_r, tab1)
    out2 = jnp.reshape(hv[1], (NC, NR128, 128))
    packed = _finish(out2, W2)
    out32 = jnp.reshape(packed, (NPAD, 2 * LANES))
    return out32[:n, :W2.shape[1]]


# speedup vs baseline: 1.3874x; 1.3874x over previous
import dataclasses

import jax
import jax.numpy as jnp
from jax import lax
from jax.experimental import pallas as pl
from jax.experimental.pallas import tpu as pltpu
from jax.experimental.pallas import tpu_sc as plsc

NC = 2
NS = 16
LANES = 16
CHUNK = 400
SB = 200
NSTREAM = CHUNK // SB
NCPS = 250
RING = 6
NFULL = (NCPS // RING) * RING
ZROWS = 400
NPAD = 102400
NR128 = NPAD // 8
NCH = 1600000 // CHUNK


def _sc_compiler_params():
    cp = pltpu.CompilerParams()
    fields = pltpu.CompilerParams.__dataclass_fields__
    if "needs_layout_passes" in fields:
        cp = dataclasses.replace(cp, needs_layout_passes=False)
    if "use_tc_tiling_on_sc" in fields:
        cp = dataclasses.replace(cp, use_tc_tiling_on_sc=False)
    return cp


def _gcn_body(dst_hbm, src_hbm, w_hbm, tab_hbm, h1_hbm, out_hbm,
              dst_v, src_v, w_v, rows, acc,
              si0, si1, si2, sg0, sg1, ss0, ss1):
    c = lax.axis_index("c")
    s = lax.axis_index("s")
    sems_i = (si0, si1, si2)
    sems_g = (sg0, sg1)
    sems_s = (ss0, ss1)
    rows_per_sub = NPAD // NS

    def zero_acc_stripe():
        @plsc.parallel_loop(0, ZROWS, unroll=4)
        def _zero_buf(i):
            rows[0, i, :] = jnp.zeros((LANES,), jnp.float32)

        @pl.loop(0, rows_per_sub // ZROWS)
        def _zero_acc(k):
            pltpu.sync_copy(rows.at[0].at[pl.ds(0, ZROWS)],
                            acc.at[pl.ds(s * rows_per_sub + k * ZROWS, ZROWS)])

    def chunk_id(ci):
        return ci * NS + s

    def fire_idx(ci, t):
        ch = chunk_id(ci)
        pltpu.async_copy(dst_hbm.at[ch], dst_v.at[t], sems_i[t])
        pltpu.async_copy(src_hbm.at[ch], src_v.at[t], sems_i[t])
        pltpu.async_copy(w_hbm.at[ch], w_v.at[t], sems_i[t])

    def wait_idx(ci, t):
        ch = chunk_id(ci)
        pltpu.make_async_copy(dst_hbm.at[ch], dst_v.at[t], sems_i[t]).wait()
        pltpu.make_async_copy(src_hbm.at[ch], src_v.at[t], sems_i[t]).wait()
        pltpu.make_async_copy(w_hbm.at[ch], w_v.at[t], sems_i[t]).wait()

    def fire_scatter(t, r):
        for j in range(NSTREAM):
            pltpu.async_copy(rows.at[r].at[pl.ds(j * SB, SB)],
                             acc.at[dst_v.at[t].at[j]], sems_s[r], add=True)

    def wait_scatter(t, r):
        for j in range(NSTREAM):
            pltpu.make_async_copy(rows.at[r].at[pl.ds(j * SB, SB)],
                                  acc.at[dst_v.at[t].at[j]],
                                  sems_s[r]).wait()

    def run_pass(tab, do_relu, wb):
        def fire_gather(t, r):
            for j in range(NSTREAM):
                pltpu.async_copy(tab.at[src_v.at[t].at[j]],
                                 rows.at[r].at[pl.ds(j * SB, SB)], sems_g[r])

        def wait_gather(t, r):
            for j in range(NSTREAM):
                pltpu.make_async_copy(tab.at[src_v.at[t].at[j]],
                                      rows.at[r].at[pl.ds(j * SB, SB)],
                                      sems_g[r]).wait()

        def multiply(t, r):
            rv = rows.at[r]
            wv = w_v.at[t]

            @plsc.parallel_loop(0, CHUNK // LANES)
            def _grp(g):
                base = g * LANES
                wgrp = wv[pl.ds(base, LANES)]
                for i in range(LANES):
                    wb = lax.gather(
                        wgrp, jnp.full((LANES, 1), i, jnp.int32),
                        dimension_numbers=lax.GatherDimensionNumbers(
                            offset_dims=(), collapsed_slice_dims=(0,),
                            start_index_map=(0,)),
                        slice_sizes=(1,),
                        mode=lax.GatherScatterMode.PROMISE_IN_BOUNDS)
                    if do_relu:
                        rows[r, base + i, :] = (
                            jnp.maximum(rv[base + i, :], 0.0) * wb)
                    else:
                        rows[r, base + i, :] = rv[base + i, :] * wb

        def ring_step(ci, k):
            t, tp1, tp2 = k % 3, (k + 1) % 3, (k + 2) % 3
            r, rp1 = k % 2, (k + 1) % 2

            @pl.when(ci > 0)
            def _():
                wait_scatter(tp2, rp1)

            @pl.when(ci + 1 < NCPS)
            def _():
                wait_idx(ci + 1, tp1)
                fire_gather(tp1, rp1)

            @pl.when(ci + 2 < NCPS)
            def _():
                fire_idx(ci + 2, tp2)

            wait_gather(t, r)
            multiply(t, r)
            fire_scatter(t, r)

        fire_idx(0, 0)
        fire_idx(1, 1)
        wait_idx(0, 0)
        fire_gather(0, 0)

        @pl.loop(0, NFULL // RING)
        def _ring(rb):
            base = rb * RING
            for k in range(RING):
                ring_step(base + k, k)

        for ci in range(NFULL, NCPS):
            ring_step(ci, ci % RING)

        wait_scatter((NCPS - 1) % 3, (NCPS - 1) % 2)

        plsc.subcore_barrier()
        wb()

    def wb_h1():
        pltpu.sync_copy(
            acc.at[pl.ds(s * rows_per_sub, rows_per_sub)],
            h1_hbm.at[c].at[pl.ds(s * rows_per_sub, rows_per_sub)])

    def wb_out():
        pltpu.sync_copy(
            acc.at[pl.ds(s * rows_per_sub, rows_per_sub)],
            out_hbm.at[c].at[pl.ds(s * rows_per_sub, rows_per_sub)])

    zero_acc_stripe()
    plsc.subcore_barrier()
    run_pass(tab_hbm.at[c], False, wb_h1)
    zero_acc_stripe()
    plsc.subcore_barrier()
    run_pass(h1_hbm.at[c], True, wb_out)


def _gcn_sc(dst_r, src_r, w_r, tab):
    mesh = plsc.VectorSubcoreMesh(core_axis_name="c", subcore_axis_name="s")
    k = pl.kernel(
        _gcn_body,
        out_type=(jax.ShapeDtypeStruct((NC, NPAD, LANES), jnp.float32),
                  jax.ShapeDtypeStruct((NC, NPAD, LANES), jnp.float32)),
        mesh=mesh,
        scratch_types=[
            pltpu.VMEM((3, NSTREAM, SB), jnp.int32),
            pltpu.VMEM((3, NSTREAM, SB), jnp.int32),
            pltpu.VMEM((3, CHUNK), jnp.float32),
            pltpu.VMEM((2, CHUNK, LANES), jnp.float32),
            pltpu.VMEM_SHARED((NPAD, LANES), jnp.float32),
            pltpu.SemaphoreType.DMA,
            pltpu.SemaphoreType.DMA,
            pltpu.SemaphoreType.DMA,
            pltpu.SemaphoreType.DMA,
            pltpu.SemaphoreType.DMA,
            pltpu.SemaphoreType.DMA,
            pltpu.SemaphoreType.DMA,
        ],
        compiler_params=_sc_compiler_params(),
    )
    return k(dst_r, src_r, w_r, tab)


def _finish_body(x_ref, m_ref, o_ref):
    hp = lax.Precision.HIGHEST
    o_ref[...] = (jnp.dot(x_ref[0], m_ref[0], precision=hp)
                  + jnp.dot(x_ref[1], m_ref[1], precision=hp))


def _finish(v128, w2, n):
    ncls = w2.shape[1]
    eye8 = jnp.eye(8, dtype=jnp.float32)
    mats = jnp.stack([jnp.kron(eye8, w2[:LANES, :]),
                      jnp.kron(eye8, w2[LANES:, :])])
    bn = 3200
    grid = NR128 // bn
    packed = pl.pallas_call(
        _finish_body,
        grid=(grid,),
        in_specs=[
            pl.BlockSpec((NC, bn, 128), lambda i: (0, i, 0)),
            pl.BlockSpec((2, 128, 8 * ncls), lambda i: (0, 0, 0)),
        ],
        out_specs=pl.BlockSpec((bn, 8 * ncls), lambda i: (i, 0)),
        out_shape=jax.ShapeDtypeStruct((NR128, 8 * ncls), jnp.float32),
    )(v128, mats)
    return jnp.reshape(packed, (NPAD, ncls))[:n]


def kernel(edge_index, edge_weight, W1, W2):
    n = W1.shape[0]
    dst_r = edge_index[0].astype(jnp.int32).reshape(NCH, NSTREAM, SB)
    src_r = edge_index[1].astype(jnp.int32).reshape(NCH, NSTREAM, SB)
    w_r = edge_weight.reshape(NCH, CHUNK)
    tab1 = W1.reshape(n, NC, LANES).transpose(1, 0, 2)
    _, out2 = _gcn_sc(dst_r, src_r, w_r, tab1)
    return _finish(jnp.reshape(out2, (NC, NR128, 128)), W2, n)


# speedup vs baseline: 1.4474x; 1.0432x over previous
import dataclasses

import jax
import jax.numpy as jnp
from jax import lax
from jax.experimental import pallas as pl
from jax.experimental.pallas import tpu as pltpu
from jax.experimental.pallas import tpu_sc as plsc

NC = 2
NS = 16
LANES = 16
CHUNK = 400
SB = 200
NSTREAM = CHUNK // SB
NCPS = 250
RING = 6
NFULL = (NCPS // RING) * RING
ZROWS = 400
NPAD = 102400
NR128 = NPAD // 8
NCH = 1600000 // CHUNK


def _sc_compiler_params():
    cp = pltpu.CompilerParams()
    fields = pltpu.CompilerParams.__dataclass_fields__
    if "needs_layout_passes" in fields:
        cp = dataclasses.replace(cp, needs_layout_passes=False)
    if "use_tc_tiling_on_sc" in fields:
        cp = dataclasses.replace(cp, use_tc_tiling_on_sc=False)
    return cp


def _gcn_body(ei_hbm, w_hbm, tab_hbm, out_hbm,
              dst_v, src_v, w_v, rows, acc,
              si0, si1, si2, sg0, sg1, ss0, ss1):
    c = lax.axis_index("c")
    s = lax.axis_index("s")
    sems_i = (si0, si1, si2)
    sems_g = (sg0, sg1)
    sems_s = (ss0, ss1)
    rows_per_sub = NPAD // NS

    def zero_acc_stripe():
        @plsc.parallel_loop(0, ZROWS, unroll=4)
        def _zero_buf(i):
            rows[0, i, :] = jnp.zeros((LANES,), jnp.float32)

        @pl.loop(0, rows_per_sub // ZROWS)
        def _zero_acc(k):
            pltpu.sync_copy(rows.at[0].at[pl.ds(0, ZROWS)],
                            acc.at[pl.ds(s * rows_per_sub + k * ZROWS, ZROWS)])

    def chunk_id(ci):
        return ci * NS + s

    def idx_copies(ci, t):
        q = chunk_id(ci) * CHUNK
        cps = [(w_hbm.at[pl.ds(q, CHUNK)], w_v.at[t])]
        for j in range(NSTREAM):
            cps.append((ei_hbm.at[0].at[pl.ds(q + j * SB, SB)],
                        dst_v.at[t].at[j]))
            cps.append((ei_hbm.at[1].at[pl.ds(q + j * SB, SB)],
                        src_v.at[t].at[j]))
        return cps

    def fire_idx(ci, t):
        for src, dst in idx_copies(ci, t):
            pltpu.async_copy(src, dst, sems_i[t])

    def wait_idx(ci, t):
        for src, dst in idx_copies(ci, t):
            pltpu.make_async_copy(src, dst, sems_i[t]).wait()

    def fire_scatter(t, r):
        for j in range(NSTREAM):
            pltpu.async_copy(rows.at[r].at[pl.ds(j * SB, SB)],
                             acc.at[dst_v.at[t].at[j]], sems_s[r], add=True)

    def wait_scatter(t, r):
        for j in range(NSTREAM):
            pltpu.make_async_copy(rows.at[r].at[pl.ds(j * SB, SB)],
                                  acc.at[dst_v.at[t].at[j]],
                                  sems_s[r]).wait()

    def run_pass(tab, do_relu):
        def fire_gather(t, r):
            for j in range(NSTREAM):
                pltpu.async_copy(tab.at[src_v.at[t].at[j]],
                                 rows.at[r].at[pl.ds(j * SB, SB)], sems_g[r])

        def wait_gather(t, r):
            for j in range(NSTREAM):
                pltpu.make_async_copy(tab.at[src_v.at[t].at[j]],
                                      rows.at[r].at[pl.ds(j * SB, SB)],
                                      sems_g[r]).wait()

        def multiply(t, r):
            rv = rows.at[r]
            wv = w_v.at[t]

            @plsc.parallel_loop(0, CHUNK // LANES)
            def _grp(g):
                base = g * LANES
                wgrp = wv[pl.ds(base, LANES)]
                for i in range(LANES):
                    wb = lax.gather(
                        wgrp, jnp.full((LANES, 1), i, jnp.int32),
                        dimension_numbers=lax.GatherDimensionNumbers(
                            offset_dims=(), collapsed_slice_dims=(0,),
                            start_index_map=(0,)),
                        slice_sizes=(1,),
                        mode=lax.GatherScatterMode.PROMISE_IN_BOUNDS)
                    if do_relu:
                        rows[r, base + i, :] = (
                            jnp.maximum(rv[base + i, :], 0.0) * wb)
                    else:
                        rows[r, base + i, :] = rv[base + i, :] * wb

        def ring_step(ci, k):
            t, tp1, tp2 = k % 3, (k + 1) % 3, (k + 2) % 3
            r, rp1 = k % 2, (k + 1) % 2

            @pl.when(ci > 0)
            def _():
                wait_scatter(tp2, rp1)

            @pl.when(ci + 1 < NCPS)
            def _():
                wait_idx(ci + 1, tp1)
                fire_gather(tp1, rp1)

            @pl.when(ci + 2 < NCPS)
            def _():
                fire_idx(ci + 2, tp2)

            wait_gather(t, r)
            multiply(t, r)
            fire_scatter(t, r)

        fire_idx(0, 0)
        fire_idx(1, 1)
        wait_idx(0, 0)
        fire_gather(0, 0)

        @pl.loop(0, NFULL // RING)
        def _ring(rb):
            base = rb * RING
            for k in range(RING):
                ring_step(base + k, k)

        for ci in range(NFULL, NCPS):
            ring_step(ci, ci % RING)

        wait_scatter((NCPS - 1) % 3, (NCPS - 1) % 2)

        plsc.subcore_barrier()
        pltpu.sync_copy(
            acc.at[pl.ds(s * rows_per_sub, rows_per_sub)],
            out_hbm.at[c].at[pl.ds(s * rows_per_sub, rows_per_sub)])

    zero_acc_stripe()
    plsc.subcore_barrier()
    run_pass(tab_hbm.at[c], False)
    zero_acc_stripe()
    plsc.subcore_barrier()
    run_pass(out_hbm.at[c], True)


def _gcn_sc(ei, w, tab):
    mesh = plsc.VectorSubcoreMesh(core_axis_name="c", subcore_axis_name="s")
    k = pl.kernel(
        _gcn_body,
        out_type=jax.ShapeDtypeStruct((NC, NPAD, LANES), jnp.float32),
        mesh=mesh,
        scratch_types=[
            pltpu.VMEM((3, NSTREAM, SB), jnp.int32),
            pltpu.VMEM((3, NSTREAM, SB), jnp.int32),
            pltpu.VMEM((3, CHUNK), jnp.float32),
            pltpu.VMEM((2, CHUNK, LANES), jnp.float32),
            pltpu.VMEM_SHARED((NPAD, LANES), jnp.float32),
            pltpu.SemaphoreType.DMA,
            pltpu.SemaphoreType.DMA,
            pltpu.SemaphoreType.DMA,
            pltpu.SemaphoreType.DMA,
            pltpu.SemaphoreType.DMA,
            pltpu.SemaphoreType.DMA,
            pltpu.SemaphoreType.DMA,
        ],
        compiler_params=_sc_compiler_params(),
    )
    return k(ei, w, tab)


def _finish_body(x_ref, m_ref, o_ref):
    hp = lax.Precision.HIGHEST
    o_ref[...] = (jnp.dot(x_ref[0], m_ref[0], precision=hp)
                  + jnp.dot(x_ref[1], m_ref[1], precision=hp))


def _finish(v128, w2, n):
    ncls = w2.shape[1]
    eye8 = jnp.eye(8, dtype=jnp.float32)
    mats = jnp.stack([jnp.kron(eye8, w2[:LANES, :]),
                      jnp.kron(eye8, w2[LANES:, :])])
    bn = 3200
    grid = NR128 // bn
    packed = pl.pallas_call(
        _finish_body,
        grid=(grid,),
        in_specs=[
            pl.BlockSpec((NC, bn, 128), lambda i: (0, i, 0)),
            pl.BlockSpec((2, 128, 8 * ncls), lambda i: (0, 0, 0)),
        ],
        out_specs=pl.BlockSpec((bn, 8 * ncls), lambda i: (i, 0)),
        out_shape=jax.ShapeDtypeStruct((NR128, 8 * ncls), jnp.float32),
    )(v128, mats)
    return jnp.reshape(packed, (NPAD, ncls))[:n]


def kernel(edge_index, edge_weight, W1, W2):
    n = W1.shape[0]
    tab1 = W1.reshape(n, NC, LANES).transpose(1, 0, 2)
    out2 = _gcn_sc(edge_index.astype(jnp.int32), edge_weight, tab1)
    return _finish(jnp.reshape(out2, (NC, NR128, 128)), W2, n)
